# Initial kernel scaffold; baseline (speedup 1.0000x reference)
import jax
import jax.numpy as jnp
from jax.experimental import pallas as pl


def kernel(emb_users, emb_items, edge_index):
    raise NotImplementedError("write your pallas kernel here")


# speedup vs baseline: 8.0668x; 8.0668x over previous
import functools

import jax
import jax.numpy as jnp
from jax import lax
from jax.experimental import pallas as pl
from jax.experimental.pallas import tpu as pltpu
from jax.experimental.pallas import tpu_sc as plsc

NU = 25000
NI = 25000
N = NU + NI
D = 64
E = 800000
L = 4

NC = 2
NS = 16
HALF = 25600
NP = NC * HALF
PT = HALF // NS

CH = 128
EP = 802816
NCHUNK = EP // CH
KPT = NCHUNK // NS

DR = 256
SDR = 2048
ACC_ROWS = HALF + DR
DEG_WORDS = HALF + SDR
ZPT = ACC_ROWS // NS
DPT = DEG_WORDS // NS

_mesh = plsc.VectorSubcoreMesh(
    core_axis_name="c", subcore_axis_name="s", num_cores=NC, num_subcores=NS
)


def _rsqrt16(d):
    nz = d > 0.0
    x = jnp.where(nz, d, 1.0)
    i = lax.bitcast_convert_type(x, jnp.int32)
    i = jnp.int32(0x5F3759DF) - lax.shift_right_logical(i, 1)
    y = lax.bitcast_convert_type(i, jnp.float32)
    for _ in range(3):
        y = y * (1.5 - 0.5 * x * y * y)
    return jnp.where(nz, y, 0.0)


def _local_dst(v, c, g, chunk, spread):
    loc = v - c * HALF
    m = (loc >= 0) & (loc < HALF)
    lane = jnp.arange(16, dtype=jnp.int32)
    dummy = HALF + ((g * 16 + lane + chunk * CH) & (spread - 1))
    return jnp.where(m, loc, dummy)


@functools.partial(
    pl.kernel,
    out_type=(
        jax.ShapeDtypeStruct((NP,), jnp.float32),
        jax.ShapeDtypeStruct((NP, D), jnp.float32),
    ),
    mesh=_mesh,
    compiler_params=pltpu.CompilerParams(use_tc_tiling_on_sc=False),
    scratch_types=[
        pltpu.VMEM_SHARED((DEG_WORDS,), jnp.float32),
        pltpu.VMEM((SDR,), jnp.float32),
        pltpu.VMEM((CH,), jnp.int32),
        pltpu.VMEM((CH,), jnp.int32),
        pltpu.VMEM((CH,), jnp.float32),
        pltpu.VMEM((PT,), jnp.float32),
        pltpu.VMEM((16, D), jnp.float32),
        pltpu.VMEM((16, D), jnp.float32),
    ],
)
def _deg_prep_kernel(dst_hbm, emb_hbm, dis_hbm, u0_hbm,
                     deg_sh, zeros_v, idx_d, idx_l, ones_v, dis_t, ebuf, ubuf):
    c = lax.axis_index("c")
    s = lax.axis_index("s")

    @pl.loop(0, SDR // 16)
    def _(g):
        zeros_v[pl.ds(g * 16, 16)] = jnp.zeros((16,), jnp.float32)

    @pl.loop(0, CH // 16)
    def _(g):
        ones_v[pl.ds(g * 16, 16)] = jnp.ones((16,), jnp.float32)

    pltpu.sync_copy(zeros_v.at[pl.ds(0, DPT)], deg_sh.at[pl.ds(s * DPT, DPT)])
    plsc.subcore_barrier()

    @pl.loop(0, KPT)
    def _(k):
        chunk = s + k * NS
        pltpu.sync_copy(dst_hbm.at[pl.ds(chunk * CH, CH)], idx_d)

        @pl.loop(0, CH // 16)
        def _(g):
            v = idx_d[pl.ds(g * 16, 16)]
            idx_l[pl.ds(g * 16, 16)] = _local_dst(v, c, g, chunk, SDR)

        pltpu.sync_copy(ones_v, deg_sh.at[idx_l], add=True)

    plsc.subcore_barrier()

    gbase = c * HALF + s * PT
    lbase = s * PT
    pltpu.sync_copy(deg_sh.at[pl.ds(lbase, PT)], dis_t)

    @pl.loop(0, PT // 16)
    def _(k):
        dis_t[pl.ds(k * 16, 16)] = _rsqrt16(dis_t[pl.ds(k * 16, 16)])

    pltpu.sync_copy(dis_t, dis_hbm.at[pl.ds(gbase, PT)])

    @pl.loop(0, PT // 16)
    def _(k):
        pltpu.sync_copy(emb_hbm.at[pl.ds(gbase + k * 16, 16)], ebuf)
        dv = dis_t[pl.ds(k * 16, 16)]
        for r in range(16):
            b = jnp.full((16,), dv[r], jnp.float32)
            for q in range(D // 16):
                ubuf[r, pl.ds(q * 16, 16)] = ebuf[r, pl.ds(q * 16, 16)] * b

        pltpu.sync_copy(ubuf, u0_hbm.at[pl.ds(gbase + k * 16, 16)])


def _make_layer_kernel(want_u):
    outs = [jax.ShapeDtypeStruct((NP, D), jnp.float32)]
    if want_u:
        outs.append(jax.ShapeDtypeStruct((NP, D), jnp.float32))

    def body(u_hbm, src_hbm, dst_hbm, dis_hbm, *rest):
        if want_u:
            x_hbm, uo_hbm = rest[0], rest[1]
            scratch = rest[2:]
        else:
            x_hbm = rest[0]
            uo_hbm = None
            scratch = rest[1:]
        (acc_sh, zbuf, idx_s0, idx_s1, idx_d0, idx_d1, idx_l,
         rows0, rows1, dis_t, arow, xbuf, ubuf, gsem0, gsem1) = scratch

        c = lax.axis_index("c")
        s = lax.axis_index("s")

        @pl.loop(0, 16)
        def _(r):
            @pl.loop(0, D // 16)
            def _(q):
                zbuf[r, pl.ds(q * 16, 16)] = jnp.zeros((16,), jnp.float32)

        @pl.loop(0, ZPT // 16)
        def _(k):
            pltpu.sync_copy(zbuf, acc_sh.at[pl.ds(s * ZPT + k * 16, 16)])

        plsc.subcore_barrier()

        def load_idx(k, idx_s, idx_d):
            chunk = s + k * NS
            pltpu.sync_copy(src_hbm.at[pl.ds(chunk * CH, CH)], idx_s)
            pltpu.sync_copy(dst_hbm.at[pl.ds(chunk * CH, CH)], idx_d)

        def start_gather(idx_s, rows, sem):
            pltpu.async_copy(u_hbm.at[idx_s], rows, sem)

        def finish(k, idx_s, idx_d, rows, sem):
            pltpu.make_async_copy(u_hbm.at[idx_s], rows, sem).wait()
            chunk = s + k * NS

            @pl.loop(0, CH // 16)
            def _(g):
                v = idx_d[pl.ds(g * 16, 16)]
                idx_l[pl.ds(g * 16, 16)] = _local_dst(v, c, g, chunk, DR)

            pltpu.sync_copy(rows, acc_sh.at[idx_l], add=True)

        load_idx(0, idx_s0, idx_d0)
        start_gather(idx_s0, rows0, gsem0)

        @pl.loop(0, KPT // 2)
        def _(j):
            k0 = 2 * j
            load_idx(k0 + 1, idx_s1, idx_d1)
            start_gather(idx_s1, rows1, gsem1)
            finish(k0, idx_s0, idx_d0, rows0, gsem0)

            @pl.when(j < KPT // 2 - 1)
            def _():
                load_idx(k0 + 2, idx_s0, idx_d0)
                start_gather(idx_s0, rows0, gsem0)

            finish(k0 + 1, idx_s1, idx_d1, rows1, gsem1)

        plsc.subcore_barrier()

        gbase = c * HALF + s * PT
        lbase = s * PT
        pltpu.sync_copy(dis_hbm.at[pl.ds(gbase, PT)], dis_t)

        @pl.loop(0, PT // 16)
        def _(k):
            pltpu.sync_copy(acc_sh.at[pl.ds(lbase + k * 16, 16)], arow)
            dv = dis_t[pl.ds(k * 16, 16)]
            for r in range(16):
                b = jnp.full((16,), dv[r], jnp.float32)
                for q in range(D // 16):
                    xv = arow[r, pl.ds(q * 16, 16)] * b
                    xbuf[r, pl.ds(q * 16, 16)] = xv
                    if want_u:
                        ubuf[r, pl.ds(q * 16, 16)] = xv * b

            pltpu.sync_copy(xbuf, x_hbm.at[pl.ds(gbase + k * 16, 16)])
            if want_u:
                pltpu.sync_copy(ubuf, uo_hbm.at[pl.ds(gbase + k * 16, 16)])

    return pl.kernel(
        body,
        out_type=tuple(outs),
        mesh=_mesh,
        compiler_params=pltpu.CompilerParams(use_tc_tiling_on_sc=False),
        scratch_types=[
            pltpu.VMEM_SHARED((ACC_ROWS, D), jnp.float32),
            pltpu.VMEM((16, D), jnp.float32),
            pltpu.VMEM((CH,), jnp.int32),
            pltpu.VMEM((CH,), jnp.int32),
            pltpu.VMEM((CH,), jnp.int32),
            pltpu.VMEM((CH,), jnp.int32),
            pltpu.VMEM((CH,), jnp.int32),
            pltpu.VMEM((CH, D), jnp.float32),
            pltpu.VMEM((CH, D), jnp.float32),
            pltpu.VMEM((PT,), jnp.float32),
            pltpu.VMEM((16, D), jnp.float32),
            pltpu.VMEM((16, D), jnp.float32),
            pltpu.VMEM((16, D), jnp.float32),
            pltpu.SemaphoreType.DMA,
            pltpu.SemaphoreType.DMA,
        ],
    )


_layer_mid = _make_layer_kernel(want_u=True)
_layer_last = _make_layer_kernel(want_u=False)


_BLK = 512


def _avg_body(e_ref, x1_ref, x2_ref, x3_ref, x4_ref, o_ref):
    o_ref[...] = (
        e_ref[...] + x1_ref[...] + x2_ref[...] + x3_ref[...] + x4_ref[...]
    ) * jnp.float32(1.0 / (L + 1) ** 2)


_avg_kernel = pl.pallas_call(
    _avg_body,
    out_shape=jax.ShapeDtypeStruct((NP, D), jnp.float32),
    grid=(NP // _BLK,),
    in_specs=[pl.BlockSpec((_BLK, D), lambda i: (i, 0))] * 5,
    out_specs=pl.BlockSpec((_BLK, D), lambda i: (i, 0)),
)


def kernel(emb_users, emb_items, edge_index):
    emb = jnp.concatenate([emb_users, emb_items], axis=0)
    emb = jnp.pad(emb, ((0, NP - N), (0, 0)))
    src = jnp.pad(edge_index[0], (0, EP - E))
    dst = jnp.pad(edge_index[1], (0, EP - E),
                  constant_values=jnp.int32(NP))

    dis, u = _deg_prep_kernel(dst, emb)
    xs = []
    for l in range(L):
        if l < L - 1:
            x, u = _layer_mid(u, src, dst, dis)
        else:
            (x,) = _layer_last(u, src, dst, dis)
        xs.append(x)

    out = _avg_kernel(emb, *xs)
    return (out[:NU], emb_users, out[NU:N], emb_items)


# speedup vs baseline: 26.2086x; 3.2489x over previous
import functools

import jax
import jax.numpy as jnp
from jax import lax
from jax.experimental import pallas as pl
from jax.experimental.pallas import tpu as pltpu
from jax.experimental.pallas import tpu_sc as plsc

NU = 25000
NI = 25000
N = NU + NI
D = 64
E = 800000
L = 4

NC = 2
NS = 16
W = NC * NS
HALF = 25600
NP = NC * HALF
PT = HALF // NS

CH = 128
EP = 802816
NCHUNK = EP // CH
KPT = NCHUNK // NS

BLKC = 14
NBLK = KPT // BLKC
BLKE = BLKC * CH

DR = 256
ACC_ROWS = HALF + DR
DEG_WORDS = HALF + DR
ZPT = ACC_ROWS // NS
DPT = DEG_WORDS // NS

GRP = 6
SLOT = KPT * CH + GRP * CH
SLOTR = SLOT // CH
LBC = 3
LBE = LBC * CH

_mesh = plsc.VectorSubcoreMesh(
    core_axis_name="c", subcore_axis_name="s", num_cores=NC, num_subcores=NS
)
_params = pltpu.CompilerParams(
    use_tc_tiling_on_sc=False, needs_layout_passes=False
)


def _rsqrt16(d):
    nz = d > 0.0
    x = jnp.where(nz, d, 1.0)
    i = lax.bitcast_convert_type(x, jnp.int32)
    i = jnp.int32(0x5F3759DF) - lax.shift_right_logical(i, 1)
    y = lax.bitcast_convert_type(i, jnp.float32)
    for _ in range(3):
        y = y * (1.5 - 0.5 * x * y * y)
    return jnp.where(nz, y, 0.0)


@functools.partial(
    pl.kernel,
    out_type=(
        jax.ShapeDtypeStruct((NP,), jnp.float32),
        jax.ShapeDtypeStruct((NP, D), jnp.float32),
        jax.ShapeDtypeStruct((W * SLOT,), jnp.int32),
        jax.ShapeDtypeStruct((W * SLOTR, CH), jnp.int32),
        jax.ShapeDtypeStruct((W * 16,), jnp.int32),
    ),
    mesh=_mesh,
    compiler_params=_params,
    scratch_types=[
        pltpu.VMEM_SHARED((DEG_WORDS,), jnp.float32),
        pltpu.VMEM((SLOT,), jnp.int32),
        pltpu.VMEM((SLOTR, CH), jnp.int32),
        pltpu.VMEM((BLKE,), jnp.int32),
        pltpu.VMEM((BLKE,), jnp.int32),
        pltpu.VMEM((BLKE,), jnp.int32),
        pltpu.VMEM((BLKE,), jnp.int32),
        pltpu.VMEM((CH,), jnp.float32),
        pltpu.VMEM((DPT,), jnp.float32),
        pltpu.VMEM((16,), jnp.int32),
        pltpu.VMEM((PT,), jnp.float32),
        pltpu.VMEM((64, D), jnp.float32),
        pltpu.VMEM((64, D), jnp.float32),
        pltpu.SemaphoreType.DMA,
        pltpu.SemaphoreType.DMA,
    ],
)
def _prep_kernel(src_hbm, dst_hbm, emb_hbm,
                 dis_hbm, u0_hbm, psrc_hbm, pdst_hbm, pcnt_hbm,
                 deg_sh, ps_loc, pd_loc, sb0, sb1, db0, db1, ones_v,
                 zeros_v, cnt_v, dis_t, ebuf, ubuf, sem0, sem1):
    c = lax.axis_index("c")
    s = lax.axis_index("s")
    w = c * NS + s
    lane = jnp.arange(16, dtype=jnp.int32)
    tile_e = s * KPT * CH

    @pl.loop(0, DPT // 16)
    def _(g):
        zeros_v[pl.ds(g * 16, 16)] = jnp.zeros((16,), jnp.float32)

    @pl.loop(0, CH // 16)
    def _(g):
        ones_v[pl.ds(g * 16, 16)] = jnp.ones((16,), jnp.float32)

    pltpu.sync_copy(zeros_v, deg_sh.at[pl.ds(s * DPT, DPT)])
    plsc.subcore_barrier()

    def issue(b, sb, db, sem):
        pltpu.async_copy(src_hbm.at[pl.ds(tile_e + b * BLKE, BLKE)], sb, sem)
        pltpu.async_copy(dst_hbm.at[pl.ds(tile_e + b * BLKE, BLKE)], db, sem)

    def wait(sb, db, sem):
        pltpu.make_async_copy(src_hbm.at[pl.ds(0, BLKE)], sb, sem).wait()
        pltpu.make_async_copy(dst_hbm.at[pl.ds(0, BLKE)], db, sem).wait()

    def do_block(sb, db, cnt0):
        @pl.loop(0, BLKE // 16, init_carry=cnt0)
        def cnt(g, cnt):
            v = db[pl.ds(g * 16, 16)]
            sv = sb[pl.ds(g * 16, 16)]
            loc = v - c * HALF
            m = (loc >= 0) & (loc < HALF)
            pos = plsc.cumsum(m.astype(jnp.int32))
            tgt = cnt + pos - 1
            plsc.store_scatter(
                pd_loc,
                [lax.shift_right_logical(tgt, 7), tgt & (CH - 1)],
                loc, mask=m)
            plsc.store_scatter(ps_loc, [tgt], sv, mask=m)
            return cnt + pos[15]

        return cnt

    issue(0, sb0, db0, sem0)

    @pl.loop(0, NBLK // 2, init_carry=jnp.int32(0))
    def cnt(i, cnt):
        issue(2 * i + 1, sb1, db1, sem1)
        wait(sb0, db0, sem0)
        cnt = do_block(sb0, db0, cnt)

        @pl.when(i < NBLK // 2 - 1)
        def _():
            issue(2 * i + 2, sb0, db0, sem0)

        wait(sb1, db1, sem1)
        return do_block(sb1, db1, cnt)

    for g in range(GRP * CH // 16):
        t = cnt + g * 16 + lane
        plsc.store_scatter(
            pd_loc,
            [lax.shift_right_logical(t, 7), t & (CH - 1)],
            HALF + ((g * 16 + lane) & (DR - 1)))
        plsc.store_scatter(ps_loc, [t], g * 16 + lane)

    nch = ((cnt + GRP * CH - 1) // (GRP * CH)) * GRP
    cnt_v[...] = jnp.full((16,), nch, jnp.int32)
    pltpu.sync_copy(ps_loc, psrc_hbm.at[pl.ds(w * SLOT, SLOT)])
    pltpu.sync_copy(pd_loc, pdst_hbm.at[pl.ds(w * SLOTR, SLOTR)])
    pltpu.sync_copy(cnt_v, pcnt_hbm.at[pl.ds(w * 16, 16)])

    @pl.loop(0, nch)
    def _(kk):
        pltpu.sync_copy(ones_v, deg_sh.at[pd_loc.at[kk]], add=True)

    plsc.subcore_barrier()

    gbase = c * HALF + s * PT
    pltpu.sync_copy(deg_sh.at[pl.ds(s * PT, PT)], dis_t)

    @pl.loop(0, PT // 16)
    def _(k):
        dis_t[pl.ds(k * 16, 16)] = _rsqrt16(dis_t[pl.ds(k * 16, 16)])

    pltpu.sync_copy(dis_t, dis_hbm.at[pl.ds(gbase, PT)])

    @pl.loop(0, PT // 64)
    def _(k):
        pltpu.sync_copy(emb_hbm.at[pl.ds(gbase + k * 64, 64)], ebuf)
        for h in range(4):
            dv = dis_t[pl.ds(k * 64 + h * 16, 16)]
            for r in range(16):
                b = jnp.full((16,), dv[r], jnp.float32)
                for q in range(D // 16):
                    ubuf[h * 16 + r, pl.ds(q * 16, 16)] = (
                        ebuf[h * 16 + r, pl.ds(q * 16, 16)] * b)
        pltpu.sync_copy(ubuf, u0_hbm.at[pl.ds(gbase + k * 64, 64)])


def _make_layer_kernel(want_u):
    outs = [jax.ShapeDtypeStruct((NP, D), jnp.float32)]
    if want_u:
        outs.append(jax.ShapeDtypeStruct((NP, D), jnp.float32))

    def body(u_hbm, psrc_hbm, pdst_hbm, pcnt_hbm, dis_hbm, *rest):
        if want_u:
            x_hbm, uo_hbm = rest[0], rest[1]
            scratch = rest[2:]
        else:
            x_hbm = rest[0]
            uo_hbm = None
            scratch = rest[1:]
        (acc_sh, sblk0, sblk1, dblk0, dblk1, cnt_v,
         rows0, rows1, rows2, dis64,
         isem0, isem1, gsem0, gsem1, gsem2) = scratch

        c = lax.axis_index("c")
        s = lax.axis_index("s")
        w = c * NS + s
        slot_e = w * SLOT
        slot_r = w * SLOTR

        pltpu.sync_copy(pcnt_hbm.at[pl.ds(w * 16, 16)], cnt_v)
        nch = cnt_v[...][0]
        nb2 = nch // GRP

        @pl.loop(0, CH)
        def _(r):
            @pl.loop(0, D // 16)
            def _(q):
                rows0[r, pl.ds(q * 16, 16)] = jnp.zeros((16,), jnp.float32)

        @pl.loop(0, ZPT // CH)
        def _(k):
            pltpu.sync_copy(rows0, acc_sh.at[pl.ds(s * ZPT + k * CH, CH)])

        pltpu.sync_copy(rows0.at[pl.ds(0, ZPT % CH)],
                        acc_sh.at[pl.ds(s * ZPT + (ZPT // CH) * CH,
                                        ZPT % CH)])
        plsc.subcore_barrier()

        def issue_blk(b, sb, db, sem):
            pltpu.async_copy(psrc_hbm.at[pl.ds(slot_e + b * LBE, LBE)],
                             sb, sem)
            pltpu.async_copy(pdst_hbm.at[pl.ds(slot_r + b * LBC, LBC)],
                             db, sem)

        def wait_blk(sb, db, sem):
            pltpu.make_async_copy(psrc_hbm.at[pl.ds(0, LBE)], sb, sem).wait()
            pltpu.make_async_copy(pdst_hbm.at[pl.ds(0, LBC)], db, sem).wait()

        def start_g(sb, t, rbuf, sem):
            pltpu.async_copy(u_hbm.at[sb.at[pl.ds(t * CH, CH)]], rbuf, sem)

        def fin(db, t, rbuf, sem):
            pltpu.make_async_copy(u_hbm.at[pl.ds(0, CH)], rbuf, sem).wait()
            pltpu.sync_copy(rbuf, acc_sh.at[db.at[t]], add=True)

        @pl.when(nch > 0)
        def _():
            issue_blk(0, sblk0, dblk0, isem0)
            issue_blk(1, sblk1, dblk1, isem1)
            wait_blk(sblk0, dblk0, isem0)
            start_g(sblk0, 0, rows0, gsem0)
            start_g(sblk0, 1, rows1, gsem1)

            @pl.loop(0, nb2)
            def _(i):
                last = i == nb2 - 1
                start_g(sblk0, 2, rows2, gsem2)
                fin(dblk0, 0, rows0, gsem0)

                wait_blk(sblk1, dblk1, isem1)
                start_g(sblk1, 0, rows0, gsem0)
                fin(dblk0, 1, rows1, gsem1)

                start_g(sblk1, 1, rows1, gsem1)
                fin(dblk0, 2, rows2, gsem2)

                @pl.when(~last)
                def _():
                    issue_blk(2 * i + 2, sblk0, dblk0, isem0)

                start_g(sblk1, 2, rows2, gsem2)
                fin(dblk1, 0, rows0, gsem0)

                @pl.when(~last)
                def _():
                    wait_blk(sblk0, dblk0, isem0)
                    start_g(sblk0, 0, rows0, gsem0)

                fin(dblk1, 1, rows1, gsem1)

                @pl.when(~last)
                def _():
                    start_g(sblk0, 1, rows1, gsem1)

                fin(dblk1, 2, rows2, gsem2)

                @pl.when(~last)
                def _():
                    issue_blk(2 * i + 3, sblk1, dblk1, isem1)

        plsc.subcore_barrier()

        gbase = c * HALF + s * PT
        lbase = s * PT

        @pl.loop(0, PT // 64)
        def _(k):
            pltpu.sync_copy(acc_sh.at[pl.ds(lbase + k * 64, 64)],
                            rows0.at[pl.ds(0, 64)])
            pltpu.sync_copy(dis_hbm.at[pl.ds(gbase + k * 64, 64)], dis64)
            for h in range(4):
                dv = dis64[pl.ds(h * 16, 16)]
                for r in range(16):
                    b = jnp.full((16,), dv[r], jnp.float32)
                    for q in range(D // 16):
                        xv = rows0[h * 16 + r, pl.ds(q * 16, 16)] * b
                        rows0[64 + h * 16 + r, pl.ds(q * 16, 16)] = xv
                        if want_u:
                            rows1[h * 16 + r, pl.ds(q * 16, 16)] = xv * b
            pltpu.sync_copy(rows0.at[pl.ds(64, 64)],
                            x_hbm.at[pl.ds(gbase + k * 64, 64)])
            if want_u:
                pltpu.sync_copy(rows1.at[pl.ds(0, 64)],
                                uo_hbm.at[pl.ds(gbase + k * 64, 64)])

    return pl.kernel(
        body,
        out_type=tuple(outs),
        mesh=_mesh,
        compiler_params=_params,
        scratch_types=[
            pltpu.VMEM_SHARED((ACC_ROWS, D), jnp.float32),
            pltpu.VMEM((LBE,), jnp.int32),
            pltpu.VMEM((LBE,), jnp.int32),
            pltpu.VMEM((LBC, CH), jnp.int32),
            pltpu.VMEM((LBC, CH), jnp.int32),
            pltpu.VMEM((16,), jnp.int32),
            pltpu.VMEM((CH, D), jnp.float32),
            pltpu.VMEM((CH, D), jnp.float32),
            pltpu.VMEM((CH, D), jnp.float32),
            pltpu.VMEM((64,), jnp.float32),
            pltpu.SemaphoreType.DMA,
            pltpu.SemaphoreType.DMA,
            pltpu.SemaphoreType.DMA,
            pltpu.SemaphoreType.DMA,
            pltpu.SemaphoreType.DMA,
        ],
    )


_layer_mid = _make_layer_kernel(want_u=True)
_layer_last = _make_layer_kernel(want_u=False)

_BLK = 512


def _avg_body(e_ref, x1_ref, x2_ref, x3_ref, x4_ref, o_ref):
    o_ref[...] = (
        e_ref[...] + x1_ref[...] + x2_ref[...] + x3_ref[...] + x4_ref[...]
    ) * jnp.float32(1.0 / (L + 1) ** 2)


_avg_kernel = pl.pallas_call(
    _avg_body,
    out_shape=jax.ShapeDtypeStruct((NP, D), jnp.float32),
    grid=(NP // _BLK,),
    in_specs=[pl.BlockSpec((_BLK, D), lambda i: (i, 0))] * 5,
    out_specs=pl.BlockSpec((_BLK, D), lambda i: (i, 0)),
)


def kernel(emb_users, emb_items, edge_index):
    emb = jnp.concatenate([emb_users, emb_items], axis=0)
    emb = jnp.pad(emb, ((0, NP - N), (0, 0)))
    src = jnp.pad(edge_index[0], (0, EP - E))
    dst = jnp.pad(edge_index[1], (0, EP - E),
                  constant_values=jnp.int32(NP))

    dis, u, psrc, pdst, pcnt = _prep_kernel(src, dst, emb)
    xs = []
    for l in range(L):
        if l < L - 1:
            x, u = _layer_mid(u, psrc, pdst, pcnt, dis)
        else:
            (x,) = _layer_last(u, psrc, pdst, pcnt, dis)
        xs.append(x)

    out = _avg_kernel(emb, *xs)
    return (out[:NU], emb_users, out[NU:N], emb_items)


# speedup vs baseline: 27.0135x; 1.0307x over previous
import functools

import jax
import jax.numpy as jnp
from jax import lax
from jax.experimental import pallas as pl
from jax.experimental.pallas import tpu as pltpu
from jax.experimental.pallas import tpu_sc as plsc

NU = 25000
NI = 25000
N = NU + NI
D = 64
E = 800000
L = 4

NC = 2
NS = 16
W = NC * NS
HALF = 25600
NP = NC * HALF
PT = HALF // NS

CH = 128
EP = 802816
NCHUNK = EP // CH
KPT = NCHUNK // NS

BLKC = 14
NBLK = KPT // BLKC
BLKE = BLKC * CH

DR = 256
ACC_ROWS = HALF + DR
DEG_WORDS = HALF + DR
ZPT = ACC_ROWS // NS
DPT = DEG_WORDS // NS

GRP = 6
SLOT = KPT * CH + GRP * CH
SLOTR = SLOT // CH
LBC = 3
LBE = LBC * CH

_mesh = plsc.VectorSubcoreMesh(
    core_axis_name="c", subcore_axis_name="s", num_cores=NC, num_subcores=NS
)
_params = pltpu.CompilerParams(
    use_tc_tiling_on_sc=False, needs_layout_passes=False
)


def _rsqrt16(d):
    nz = d > 0.0
    x = jnp.where(nz, d, 1.0)
    i = lax.bitcast_convert_type(x, jnp.int32)
    i = jnp.int32(0x5F3759DF) - lax.shift_right_logical(i, 1)
    y = lax.bitcast_convert_type(i, jnp.float32)
    for _ in range(3):
        y = y * (1.5 - 0.5 * x * y * y)
    return jnp.where(nz, y, 0.0)


@functools.partial(
    pl.kernel,
    out_type=(
        jax.ShapeDtypeStruct((NP,), jnp.float32),
        jax.ShapeDtypeStruct((NP,), jnp.float32),
        jax.ShapeDtypeStruct((NP, D), jnp.float32),
        jax.ShapeDtypeStruct((W * SLOT,), jnp.int32),
        jax.ShapeDtypeStruct((W * SLOTR, CH), jnp.int32),
        jax.ShapeDtypeStruct((W * 16,), jnp.int32),
    ),
    mesh=_mesh,
    compiler_params=_params,
    scratch_types=[
        pltpu.VMEM_SHARED((DEG_WORDS,), jnp.float32),
        pltpu.VMEM((SLOT,), jnp.int32),
        pltpu.VMEM((SLOTR, CH), jnp.int32),
        pltpu.VMEM((BLKE,), jnp.int32),
        pltpu.VMEM((BLKE,), jnp.int32),
        pltpu.VMEM((BLKE,), jnp.int32),
        pltpu.VMEM((BLKE,), jnp.int32),
        pltpu.VMEM((CH,), jnp.float32),
        pltpu.VMEM((DPT,), jnp.float32),
        pltpu.VMEM((16,), jnp.int32),
        pltpu.VMEM((PT,), jnp.float32),
        pltpu.VMEM((64, D), jnp.float32),
        pltpu.VMEM((64, D), jnp.float32),
        pltpu.SemaphoreType.DMA,
        pltpu.SemaphoreType.DMA,
    ],
)
def _prep_kernel(src_hbm, dst_hbm, emb_hbm,
                 dis_hbm, disinv_hbm, u0_hbm, psrc_hbm, pdst_hbm, pcnt_hbm,
                 deg_sh, ps_loc, pd_loc, sb0, sb1, db0, db1, ones_v,
                 zeros_v, cnt_v, dis_t, ebuf, ubuf, sem0, sem1):
    c = lax.axis_index("c")
    s = lax.axis_index("s")
    w = c * NS + s
    lane = jnp.arange(16, dtype=jnp.int32)
    tile_e = s * KPT * CH

    @pl.loop(0, DPT // 16)
    def _(g):
        zeros_v[pl.ds(g * 16, 16)] = jnp.zeros((16,), jnp.float32)

    @pl.loop(0, CH // 16)
    def _(g):
        ones_v[pl.ds(g * 16, 16)] = jnp.ones((16,), jnp.float32)

    pltpu.sync_copy(zeros_v, deg_sh.at[pl.ds(s * DPT, DPT)])
    plsc.subcore_barrier()

    def issue(b, sb, db, sem):
        pltpu.async_copy(src_hbm.at[pl.ds(tile_e + b * BLKE, BLKE)], sb, sem)
        pltpu.async_copy(dst_hbm.at[pl.ds(tile_e + b * BLKE, BLKE)], db, sem)

    def wait(sb, db, sem):
        pltpu.make_async_copy(src_hbm.at[pl.ds(0, BLKE)], sb, sem).wait()
        pltpu.make_async_copy(dst_hbm.at[pl.ds(0, BLKE)], db, sem).wait()

    def do_block(sb, db, cnt0):
        @pl.loop(0, BLKE // 16, init_carry=cnt0)
        def cnt(g, cnt):
            v = db[pl.ds(g * 16, 16)]
            sv = sb[pl.ds(g * 16, 16)]
            loc = v - c * HALF
            m = (loc >= 0) & (loc < HALF)
            pos = plsc.cumsum(m.astype(jnp.int32))
            tgt = cnt + pos - 1
            plsc.store_scatter(
                pd_loc,
                [lax.shift_right_logical(tgt, 7), tgt & (CH - 1)],
                loc, mask=m)
            plsc.store_scatter(ps_loc, [tgt], sv, mask=m)
            return cnt + pos[15]

        return cnt

    issue(0, sb0, db0, sem0)

    @pl.loop(0, NBLK // 2, init_carry=jnp.int32(0))
    def cnt(i, cnt):
        issue(2 * i + 1, sb1, db1, sem1)
        wait(sb0, db0, sem0)
        cnt = do_block(sb0, db0, cnt)

        @pl.when(i < NBLK // 2 - 1)
        def _():
            issue(2 * i + 2, sb0, db0, sem0)

        wait(sb1, db1, sem1)
        return do_block(sb1, db1, cnt)

    for g in range(GRP * CH // 16):
        t = cnt + g * 16 + lane
        plsc.store_scatter(
            pd_loc,
            [lax.shift_right_logical(t, 7), t & (CH - 1)],
            HALF + ((g * 16 + lane) & (DR - 1)))
        plsc.store_scatter(ps_loc, [t], g * 16 + lane)

    nch = ((cnt + GRP * CH - 1) // (GRP * CH)) * GRP
    cnt_v[...] = jnp.full((16,), nch, jnp.int32)
    pltpu.sync_copy(ps_loc, psrc_hbm.at[pl.ds(w * SLOT, SLOT)])
    pltpu.sync_copy(pd_loc, pdst_hbm.at[pl.ds(w * SLOTR, SLOTR)])
    pltpu.sync_copy(cnt_v, pcnt_hbm.at[pl.ds(w * 16, 16)])

    @pl.loop(0, nch)
    def _(kk):
        pltpu.async_copy(ones_v, deg_sh.at[pd_loc.at[kk]], sem0, add=True)

    @pl.loop(0, nch)
    def _(kk):
        pltpu.make_async_copy(ones_v, deg_sh.at[pd_loc.at[kk]], sem0).wait()

    plsc.subcore_barrier()

    gbase = c * HALF + s * PT
    pltpu.sync_copy(deg_sh.at[pl.ds(s * PT, PT)], dis_t)
    pltpu.sync_copy(deg_sh.at[pl.ds(s * PT, PT)], zeros_v.at[pl.ds(0, PT)])

    @pl.loop(0, PT // 16)
    def _(k):
        r = _rsqrt16(dis_t[pl.ds(k * 16, 16)])
        dis_t[pl.ds(k * 16, 16)] = r
        zeros_v[pl.ds(k * 16, 16)] = r * zeros_v[pl.ds(k * 16, 16)]

    pltpu.sync_copy(dis_t, dis_hbm.at[pl.ds(gbase, PT)])
    pltpu.sync_copy(zeros_v.at[pl.ds(0, PT)], disinv_hbm.at[pl.ds(gbase, PT)])

    @pl.loop(0, PT // 64)
    def _(k):
        pltpu.sync_copy(emb_hbm.at[pl.ds(gbase + k * 64, 64)], ebuf)
        for h in range(4):
            dv = dis_t[pl.ds(k * 64 + h * 16, 16)]
            for r in range(16):
                b = jnp.full((16,), dv[r], jnp.float32)
                for q in range(D // 16):
                    ubuf[h * 16 + r, pl.ds(q * 16, 16)] = (
                        ebuf[h * 16 + r, pl.ds(q * 16, 16)] * b)
        pltpu.sync_copy(ubuf, u0_hbm.at[pl.ds(gbase + k * 64, 64)])


def _make_layer_kernel():
    def body(u_hbm, psrc_hbm, pdst_hbm, pcnt_hbm, dis_hbm, uo_hbm, *scratch):
        (acc_sh, sblk0, sblk1, dblk0, dblk1, cnt_v,
         rows0, rows1, rows2, dis64,
         isem0, isem1, gsem0, gsem1, gsem2) = scratch

        c = lax.axis_index("c")
        s = lax.axis_index("s")
        w = c * NS + s
        slot_e = w * SLOT
        slot_r = w * SLOTR

        pltpu.sync_copy(pcnt_hbm.at[pl.ds(w * 16, 16)], cnt_v)
        nch = cnt_v[...][0]
        nb2 = nch // GRP

        @pl.loop(0, CH)
        def _(r):
            @pl.loop(0, D // 16)
            def _(q):
                rows0[r, pl.ds(q * 16, 16)] = jnp.zeros((16,), jnp.float32)

        @pl.loop(0, ZPT // CH)
        def _(k):
            pltpu.sync_copy(rows0, acc_sh.at[pl.ds(s * ZPT + k * CH, CH)])

        pltpu.sync_copy(rows0.at[pl.ds(0, ZPT % CH)],
                        acc_sh.at[pl.ds(s * ZPT + (ZPT // CH) * CH,
                                        ZPT % CH)])
        plsc.subcore_barrier()

        def issue_blk(b, sb, db, sem):
            pltpu.async_copy(psrc_hbm.at[pl.ds(slot_e + b * LBE, LBE)],
                             sb, sem)
            pltpu.async_copy(pdst_hbm.at[pl.ds(slot_r + b * LBC, LBC)],
                             db, sem)

        def wait_blk(sb, db, sem):
            pltpu.make_async_copy(psrc_hbm.at[pl.ds(0, LBE)], sb, sem).wait()
            pltpu.make_async_copy(pdst_hbm.at[pl.ds(0, LBC)], db, sem).wait()

        def start_g(sb, t, rbuf, sem):
            pltpu.async_copy(u_hbm.at[sb.at[pl.ds(t * CH, CH)]], rbuf, sem)

        def fin(db, t, rbuf, sem):
            pltpu.make_async_copy(u_hbm.at[pl.ds(0, CH)], rbuf, sem).wait()
            pltpu.sync_copy(rbuf, acc_sh.at[db.at[t]], add=True)

        @pl.when(nch > 0)
        def _():
            issue_blk(0, sblk0, dblk0, isem0)
            issue_blk(1, sblk1, dblk1, isem1)
            wait_blk(sblk0, dblk0, isem0)
            start_g(sblk0, 0, rows0, gsem0)
            start_g(sblk0, 1, rows1, gsem1)

            @pl.loop(0, nb2)
            def _(i):
                last = i == nb2 - 1
                start_g(sblk0, 2, rows2, gsem2)
                fin(dblk0, 0, rows0, gsem0)

                wait_blk(sblk1, dblk1, isem1)
                start_g(sblk1, 0, rows0, gsem0)
                fin(dblk0, 1, rows1, gsem1)

                start_g(sblk1, 1, rows1, gsem1)
                fin(dblk0, 2, rows2, gsem2)

                @pl.when(~last)
                def _():
                    issue_blk(2 * i + 2, sblk0, dblk0, isem0)

                start_g(sblk1, 2, rows2, gsem2)
                fin(dblk1, 0, rows0, gsem0)

                @pl.when(~last)
                def _():
                    wait_blk(sblk0, dblk0, isem0)
                    start_g(sblk0, 0, rows0, gsem0)

                fin(dblk1, 1, rows1, gsem1)

                @pl.when(~last)
                def _():
                    start_g(sblk0, 1, rows1, gsem1)

                fin(dblk1, 2, rows2, gsem2)

                @pl.when(~last)
                def _():
                    issue_blk(2 * i + 3, sblk1, dblk1, isem1)

        plsc.subcore_barrier()

        gbase = c * HALF + s * PT
        lbase = s * PT

        @pl.loop(0, PT // 64)
        def _(k):
            pltpu.sync_copy(acc_sh.at[pl.ds(lbase + k * 64, 64)],
                            rows0.at[pl.ds(0, 64)])
            pltpu.sync_copy(dis_hbm.at[pl.ds(gbase + k * 64, 64)], dis64)
            for h in range(4):
                dv = dis64[pl.ds(h * 16, 16)]
                for r in range(16):
                    b = jnp.full((16,), dv[r], jnp.float32)
                    b2 = b * b
                    for q in range(D // 16):
                        rows0[64 + h * 16 + r, pl.ds(q * 16, 16)] = (
                            rows0[h * 16 + r, pl.ds(q * 16, 16)] * b2)
            pltpu.sync_copy(rows0.at[pl.ds(64, 64)],
                            uo_hbm.at[pl.ds(gbase + k * 64, 64)])

    return pl.kernel(
        body,
        out_type=jax.ShapeDtypeStruct((NP, D), jnp.float32),
        mesh=_mesh,
        compiler_params=_params,
        scratch_types=[
            pltpu.VMEM_SHARED((ACC_ROWS, D), jnp.float32),
            pltpu.VMEM((LBE,), jnp.int32),
            pltpu.VMEM((LBE,), jnp.int32),
            pltpu.VMEM((LBC, CH), jnp.int32),
            pltpu.VMEM((LBC, CH), jnp.int32),
            pltpu.VMEM((16,), jnp.int32),
            pltpu.VMEM((CH, D), jnp.float32),
            pltpu.VMEM((CH, D), jnp.float32),
            pltpu.VMEM((CH, D), jnp.float32),
            pltpu.VMEM((64,), jnp.float32),
            pltpu.SemaphoreType.DMA,
            pltpu.SemaphoreType.DMA,
            pltpu.SemaphoreType.DMA,
            pltpu.SemaphoreType.DMA,
            pltpu.SemaphoreType.DMA,
        ],
    )


_layer_kernel = _make_layer_kernel()

_BLK = 512


def _avg_body(e_ref, u1_ref, u2_ref, u3_ref, u4_ref, di_ref, o_ref):
    usum = u1_ref[...] + u2_ref[...] + u3_ref[...] + u4_ref[...]
    o_ref[...] = (e_ref[...] + usum * di_ref[...]) * jnp.float32(
        1.0 / (L + 1) ** 2)


_avg_kernel = pl.pallas_call(
    _avg_body,
    out_shape=jax.ShapeDtypeStruct((NP, D), jnp.float32),
    grid=(NP // _BLK,),
    in_specs=[pl.BlockSpec((_BLK, D), lambda i: (i, 0))] * 5
    + [pl.BlockSpec((_BLK, 1), lambda i: (i, 0))],
    out_specs=pl.BlockSpec((_BLK, D), lambda i: (i, 0)),
)


def kernel(emb_users, emb_items, edge_index):
    emb = jnp.concatenate([emb_users, emb_items], axis=0)
    emb = jnp.pad(emb, ((0, NP - N), (0, 0)))
    src = jnp.pad(edge_index[0], (0, EP - E))
    dst = jnp.pad(edge_index[1], (0, EP - E),
                  constant_values=jnp.int32(NP))

    dis, disinv, u, psrc, pdst, pcnt = _prep_kernel(src, dst, emb)
    us = []
    for _ in range(L):
        u = _layer_kernel(u, psrc, pdst, pcnt, dis)
        us.append(u)

    out = _avg_kernel(emb, *us, disinv[:, None])
    return (out[:NU], emb_users, out[NU:N], emb_items)


# speedup vs baseline: 27.3727x; 1.0133x over previous
import functools

import jax
import jax.numpy as jnp
from jax import lax
from jax.experimental import pallas as pl
from jax.experimental.pallas import tpu as pltpu
from jax.experimental.pallas import tpu_sc as plsc

NU = 25000
NI = 25000
N = NU + NI
D = 64
E = 800000
L = 4

NC = 2
NS = 16
W = NC * NS
HALF = 25600
NP = NC * HALF
PT = HALF // NS

CH = 128
EP = 802816
NCHUNK = EP // CH
KPT = NCHUNK // NS

BLKC = 14
NBLK = KPT // BLKC
BLKE = BLKC * CH

DR = 256
ACC_ROWS = HALF + DR
DEG_WORDS = HALF + DR
ZPT = ACC_ROWS // NS
DPT = DEG_WORDS // NS

GRP = 6
SLOT = KPT * CH + GRP * CH
SLOTR = SLOT // CH
LBC = 3
LBE = LBC * CH

_mesh = plsc.VectorSubcoreMesh(
    core_axis_name="c", subcore_axis_name="s", num_cores=NC, num_subcores=NS
)
_params = pltpu.CompilerParams(
    use_tc_tiling_on_sc=False, needs_layout_passes=False
)


def _rsqrt16(d):
    nz = d > 0.0
    x = jnp.where(nz, d, 1.0)
    i = lax.bitcast_convert_type(x, jnp.int32)
    i = jnp.int32(0x5F3759DF) - lax.shift_right_logical(i, 1)
    y = lax.bitcast_convert_type(i, jnp.float32)
    for _ in range(3):
        y = y * (1.5 - 0.5 * x * y * y)
    return jnp.where(nz, y, 0.0)


@functools.partial(
    pl.kernel,
    out_type=(
        jax.ShapeDtypeStruct((NP,), jnp.float32),
        jax.ShapeDtypeStruct((NP,), jnp.float32),
        jax.ShapeDtypeStruct((NP, D), jnp.float32),
        jax.ShapeDtypeStruct((W * SLOT,), jnp.int32),
        jax.ShapeDtypeStruct((W * SLOTR, CH), jnp.int32),
        jax.ShapeDtypeStruct((W * 16,), jnp.int32),
    ),
    mesh=_mesh,
    compiler_params=_params,
    scratch_types=[
        pltpu.VMEM_SHARED((DEG_WORDS,), jnp.float32),
        pltpu.VMEM((SLOT,), jnp.int32),
        pltpu.VMEM((SLOTR, CH), jnp.int32),
        pltpu.VMEM((BLKE,), jnp.int32),
        pltpu.VMEM((BLKE,), jnp.int32),
        pltpu.VMEM((BLKE,), jnp.int32),
        pltpu.VMEM((BLKE,), jnp.int32),
        pltpu.VMEM((CH,), jnp.float32),
        pltpu.VMEM((DPT,), jnp.float32),
        pltpu.VMEM((16,), jnp.int32),
        pltpu.VMEM((PT,), jnp.float32),
        pltpu.VMEM((64, D), jnp.float32),
        pltpu.VMEM((64, D), jnp.float32),
        pltpu.SemaphoreType.DMA,
        pltpu.SemaphoreType.DMA,
    ],
)
def _prep_kernel(src_hbm, dst_hbm, emb_hbm,
                 dis_hbm, disinv_hbm, u0_hbm, psrc_hbm, pdst_hbm, pcnt_hbm,
                 deg_sh, ps_loc, pd_loc, sb0, sb1, db0, db1, ones_v,
                 zeros_v, cnt_v, dis_t, ebuf, ubuf, sem0, sem1):
    c = lax.axis_index("c")
    s = lax.axis_index("s")
    w = c * NS + s
    lane = jnp.arange(16, dtype=jnp.int32)
    tile_e = s * KPT * CH

    @pl.loop(0, DPT // 16)
    def _(g):
        zeros_v[pl.ds(g * 16, 16)] = jnp.zeros((16,), jnp.float32)

    @pl.loop(0, CH // 16)
    def _(g):
        ones_v[pl.ds(g * 16, 16)] = jnp.ones((16,), jnp.float32)

    pltpu.sync_copy(zeros_v, deg_sh.at[pl.ds(s * DPT, DPT)])
    plsc.subcore_barrier()

    def issue(b, sb, db, sem):
        pltpu.async_copy(src_hbm.at[pl.ds(tile_e + b * BLKE, BLKE)], sb, sem)
        pltpu.async_copy(dst_hbm.at[pl.ds(tile_e + b * BLKE, BLKE)], db, sem)

    def wait(sb, db, sem):
        pltpu.make_async_copy(src_hbm.at[pl.ds(0, BLKE)], sb, sem).wait()
        pltpu.make_async_copy(dst_hbm.at[pl.ds(0, BLKE)], db, sem).wait()

    def do_block(sb, db, cnt0):
        @pl.loop(0, BLKE // 16, init_carry=cnt0)
        def cnt(g, cnt):
            v = db[pl.ds(g * 16, 16)]
            sv = sb[pl.ds(g * 16, 16)]
            loc = v - c * HALF
            m = (loc >= 0) & (loc < HALF)
            pos = plsc.cumsum(m.astype(jnp.int32))
            tgt = cnt + pos - 1
            plsc.store_scatter(
                pd_loc,
                [lax.shift_right_logical(tgt, 7), tgt & (CH - 1)],
                loc, mask=m)
            plsc.store_scatter(ps_loc, [tgt], sv, mask=m)
            return cnt + pos[15]

        return cnt

    issue(0, sb0, db0, sem0)

    @pl.loop(0, NBLK // 2, init_carry=jnp.int32(0))
    def cnt(i, cnt):
        issue(2 * i + 1, sb1, db1, sem1)
        wait(sb0, db0, sem0)
        cnt = do_block(sb0, db0, cnt)

        @pl.when(i < NBLK // 2 - 1)
        def _():
            issue(2 * i + 2, sb0, db0, sem0)

        wait(sb1, db1, sem1)
        return do_block(sb1, db1, cnt)

    for g in range(GRP * CH // 16):
        t = cnt + g * 16 + lane
        plsc.store_scatter(
            pd_loc,
            [lax.shift_right_logical(t, 7), t & (CH - 1)],
            HALF + ((g * 16 + lane) & (DR - 1)))
        plsc.store_scatter(ps_loc, [t], g * 16 + lane)

    nch = ((cnt + GRP * CH - 1) // (GRP * CH)) * GRP
    cnt_v[...] = jnp.full((16,), nch, jnp.int32)
    pltpu.sync_copy(ps_loc, psrc_hbm.at[pl.ds(w * SLOT, SLOT)])
    pltpu.sync_copy(pd_loc, pdst_hbm.at[pl.ds(w * SLOTR, SLOTR)])
    pltpu.sync_copy(cnt_v, pcnt_hbm.at[pl.ds(w * 16, 16)])

    @pl.loop(0, nch)
    def _(kk):
        pltpu.async_copy(ones_v, deg_sh.at[pd_loc.at[kk]], sem0, add=True)

    @pl.loop(0, nch)
    def _(kk):
        pltpu.make_async_copy(ones_v, deg_sh.at[pd_loc.at[kk]], sem0).wait()

    plsc.subcore_barrier()

    gbase = c * HALF + s * PT
    pltpu.sync_copy(deg_sh.at[pl.ds(s * PT, PT)], dis_t)
    pltpu.sync_copy(deg_sh.at[pl.ds(s * PT, PT)], zeros_v.at[pl.ds(0, PT)])

    @pl.loop(0, PT // 16)
    def _(k):
        r = _rsqrt16(dis_t[pl.ds(k * 16, 16)])
        dis_t[pl.ds(k * 16, 16)] = r
        zeros_v[pl.ds(k * 16, 16)] = r * zeros_v[pl.ds(k * 16, 16)]

    pltpu.sync_copy(dis_t, dis_hbm.at[pl.ds(gbase, PT)])
    pltpu.sync_copy(zeros_v.at[pl.ds(0, PT)], disinv_hbm.at[pl.ds(gbase, PT)])

    @pl.loop(0, PT // 64)
    def _(k):
        pltpu.sync_copy(emb_hbm.at[pl.ds(gbase + k * 64, 64)], ebuf)
        for h in range(4):
            dv = dis_t[pl.ds(k * 64 + h * 16, 16)]
            for r in range(16):
                b = jnp.full((16,), dv[r], jnp.float32)
                for q in range(D // 16):
                    ubuf[h * 16 + r, pl.ds(q * 16, 16)] = (
                        ebuf[h * 16 + r, pl.ds(q * 16, 16)] * b)
        pltpu.sync_copy(ubuf, u0_hbm.at[pl.ds(gbase + k * 64, 64)])


def _make_layer_kernel():
    def body(u_hbm, psrc_hbm, pdst_hbm, pcnt_hbm, dis_hbm, uo_hbm, *scratch):
        (acc_sh, sblk0, sblk1, dblk0, dblk1, cnt_v,
         rows0, rows1, rows2, dis64,
         isem0, isem1, gsem0, gsem1, gsem2) = scratch

        c = lax.axis_index("c")
        s = lax.axis_index("s")
        w = c * NS + s
        slot_e = w * SLOT
        slot_r = w * SLOTR

        pltpu.sync_copy(pcnt_hbm.at[pl.ds(w * 16, 16)], cnt_v)
        nch = cnt_v[...][0]
        nb2 = nch // GRP

        @pl.loop(0, CH)
        def _(r):
            @pl.loop(0, D // 16)
            def _(q):
                rows0[r, pl.ds(q * 16, 16)] = jnp.zeros((16,), jnp.float32)

        @pl.loop(0, ZPT // CH)
        def _(k):
            pltpu.sync_copy(rows0, acc_sh.at[pl.ds(s * ZPT + k * CH, CH)])

        pltpu.sync_copy(rows0.at[pl.ds(0, ZPT % CH)],
                        acc_sh.at[pl.ds(s * ZPT + (ZPT // CH) * CH,
                                        ZPT % CH)])
        plsc.subcore_barrier()

        def issue_blk(b, sb, db, sem):
            pltpu.async_copy(psrc_hbm.at[pl.ds(slot_e + b * LBE, LBE)],
                             sb, sem)
            pltpu.async_copy(pdst_hbm.at[pl.ds(slot_r + b * LBC, LBC)],
                             db, sem)

        def wait_blk(sb, db, sem):
            pltpu.make_async_copy(psrc_hbm.at[pl.ds(0, LBE)], sb, sem).wait()
            pltpu.make_async_copy(pdst_hbm.at[pl.ds(0, LBC)], db, sem).wait()

        def start_g(sb, t, rbuf, sem):
            pltpu.async_copy(u_hbm.at[sb.at[pl.ds(t * CH, CH)]], rbuf, sem)

        def fin(db, t, rbuf, sem):
            pltpu.make_async_copy(u_hbm.at[pl.ds(0, CH)], rbuf, sem).wait()
            pltpu.sync_copy(rbuf, acc_sh.at[db.at[t]], add=True)

        @pl.when(nch > 0)
        def _():
            issue_blk(0, sblk0, dblk0, isem0)
            issue_blk(1, sblk1, dblk1, isem1)
            wait_blk(sblk0, dblk0, isem0)
            start_g(sblk0, 0, rows0, gsem0)
            start_g(sblk0, 1, rows1, gsem1)

            @pl.loop(0, nb2)
            def _(i):
                last = i == nb2 - 1
                start_g(sblk0, 2, rows2, gsem2)
                fin(dblk0, 0, rows0, gsem0)

                wait_blk(sblk1, dblk1, isem1)
                start_g(sblk1, 0, rows0, gsem0)
                fin(dblk0, 1, rows1, gsem1)

                start_g(sblk1, 1, rows1, gsem1)
                fin(dblk0, 2, rows2, gsem2)

                @pl.when(~last)
                def _():
                    issue_blk(2 * i + 2, sblk0, dblk0, isem0)

                start_g(sblk1, 2, rows2, gsem2)
                fin(dblk1, 0, rows0, gsem0)

                @pl.when(~last)
                def _():
                    wait_blk(sblk0, dblk0, isem0)
                    start_g(sblk0, 0, rows0, gsem0)

                fin(dblk1, 1, rows1, gsem1)

                @pl.when(~last)
                def _():
                    start_g(sblk0, 1, rows1, gsem1)

                fin(dblk1, 2, rows2, gsem2)

                @pl.when(~last)
                def _():
                    issue_blk(2 * i + 3, sblk1, dblk1, isem1)

        plsc.subcore_barrier()

        gbase = c * HALF + s * PT
        lbase = s * PT

        def co_read(k):
            pltpu.sync_copy(acc_sh.at[pl.ds(lbase + k * 64, 64)],
                            rows0.at[pl.ds(0, 64)])
            pltpu.sync_copy(dis_hbm.at[pl.ds(gbase + k * 64, 64)], dis64)

        def co_wait_write(stg, sem):
            pltpu.make_async_copy(stg.at[pl.ds(0, 64)],
                                  uo_hbm.at[pl.ds(gbase, 64)], sem).wait()

        def co_chunk(k, stg, sem):
            for h in range(4):
                dv = dis64[pl.ds(h * 16, 16)]
                for r in range(16):
                    b = jnp.full((16,), dv[r], jnp.float32)
                    b2 = b * b
                    for q in range(D // 16):
                        stg[h * 16 + r, pl.ds(q * 16, 16)] = (
                            rows0[h * 16 + r, pl.ds(q * 16, 16)] * b2)
            pltpu.async_copy(stg.at[pl.ds(0, 64)],
                             uo_hbm.at[pl.ds(gbase + k * 64, 64)], sem)

        @pl.loop(0, PT // 128)
        def _(j):
            k0 = 2 * j
            co_read(k0)

            @pl.when(j > 0)
            def _():
                co_wait_write(rows1, gsem0)

            co_chunk(k0, rows1, gsem0)
            co_read(k0 + 1)

            @pl.when(j > 0)
            def _():
                co_wait_write(rows2, gsem1)

            co_chunk(k0 + 1, rows2, gsem1)

        co_read(PT // 64 - 1)
        co_wait_write(rows1, gsem0)
        co_chunk(PT // 64 - 1, rows1, gsem0)
        co_wait_write(rows2, gsem1)
        co_wait_write(rows1, gsem0)

    return pl.kernel(
        body,
        out_type=jax.ShapeDtypeStruct((NP, D), jnp.float32),
        mesh=_mesh,
        compiler_params=_params,
        scratch_types=[
            pltpu.VMEM_SHARED((ACC_ROWS, D), jnp.float32),
            pltpu.VMEM((LBE,), jnp.int32),
            pltpu.VMEM((LBE,), jnp.int32),
            pltpu.VMEM((LBC, CH), jnp.int32),
            pltpu.VMEM((LBC, CH), jnp.int32),
            pltpu.VMEM((16,), jnp.int32),
            pltpu.VMEM((CH, D), jnp.float32),
            pltpu.VMEM((CH, D), jnp.float32),
            pltpu.VMEM((CH, D), jnp.float32),
            pltpu.VMEM((64,), jnp.float32),
            pltpu.SemaphoreType.DMA,
            pltpu.SemaphoreType.DMA,
            pltpu.SemaphoreType.DMA,
            pltpu.SemaphoreType.DMA,
            pltpu.SemaphoreType.DMA,
        ],
    )


_layer_kernel = _make_layer_kernel()

_BLK = 512


def _avg_body(e_ref, u1_ref, u2_ref, u3_ref, u4_ref, di_ref, o_ref):
    usum = u1_ref[...] + u2_ref[...] + u3_ref[...] + u4_ref[...]
    o_ref[...] = (e_ref[...] + usum * di_ref[...]) * jnp.float32(
        1.0 / (L + 1) ** 2)


_avg_kernel = pl.pallas_call(
    _avg_body,
    out_shape=jax.ShapeDtypeStruct((NP, D), jnp.float32),
    grid=(NP // _BLK,),
    in_specs=[pl.BlockSpec((_BLK, D), lambda i: (i, 0))] * 5
    + [pl.BlockSpec((_BLK, 1), lambda i: (i, 0))],
    out_specs=pl.BlockSpec((_BLK, D), lambda i: (i, 0)),
)


def kernel(emb_users, emb_items, edge_index):
    emb = jnp.concatenate([emb_users, emb_items], axis=0)
    emb = jnp.pad(emb, ((0, NP - N), (0, 0)))
    src = jnp.pad(edge_index[0], (0, EP - E))
    dst = jnp.pad(edge_index[1], (0, EP - E),
                  constant_values=jnp.int32(NP))

    dis, disinv, u, psrc, pdst, pcnt = _prep_kernel(src, dst, emb)
    us = []
    for _ in range(L):
        u = _layer_kernel(u, psrc, pdst, pcnt, dis)
        us.append(u)

    out = _avg_kernel(emb, *us, disinv[:, None])
    return (out[:NU], emb_users, out[NU:N], emb_items)


# speedup vs baseline: 27.5272x; 1.0056x over previous
import functools

import jax
import jax.numpy as jnp
from jax import lax
from jax.experimental import pallas as pl
from jax.experimental.pallas import tpu as pltpu
from jax.experimental.pallas import tpu_sc as plsc

NU = 25000
NI = 25000
N = NU + NI
D = 64
E = 800000
L = 4

NC = 2
NS = 16
W = NC * NS
HALF = 25600
NP = NC * HALF
PT = HALF // NS

CH = 128
EP = 802816
NCHUNK = EP // CH
KPT = NCHUNK // NS

BLKC = 14
NBLK = KPT // BLKC
BLKE = BLKC * CH

DR = 256
ACC_ROWS = HALF + DR
DEG_WORDS = HALF + DR
ZPT = ACC_ROWS // NS
DPT = DEG_WORDS // NS

GRP = 6
SLOT = KPT * CH + GRP * CH
SLOTR = SLOT // CH
LBC = 3
LBE = LBC * CH

_mesh = plsc.VectorSubcoreMesh(
    core_axis_name="c", subcore_axis_name="s", num_cores=NC, num_subcores=NS
)
_params = pltpu.CompilerParams(
    use_tc_tiling_on_sc=False, needs_layout_passes=False
)


def _rsqrt16(d):
    nz = d > 0.0
    x = jnp.where(nz, d, 1.0)
    i = lax.bitcast_convert_type(x, jnp.int32)
    i = jnp.int32(0x5F3759DF) - lax.shift_right_logical(i, 1)
    y = lax.bitcast_convert_type(i, jnp.float32)
    for _ in range(3):
        y = y * (1.5 - 0.5 * x * y * y)
    return jnp.where(nz, y, 0.0)


@functools.partial(
    pl.kernel,
    out_type=(
        jax.ShapeDtypeStruct((NP,), jnp.float32),
        jax.ShapeDtypeStruct((NP,), jnp.float32),
        jax.ShapeDtypeStruct((NP, D), jnp.float32),
        jax.ShapeDtypeStruct((W * SLOT,), jnp.int32),
        jax.ShapeDtypeStruct((W * SLOTR, CH), jnp.int32),
        jax.ShapeDtypeStruct((W * 16,), jnp.int32),
    ),
    mesh=_mesh,
    compiler_params=_params,
    scratch_types=[
        pltpu.VMEM_SHARED((DEG_WORDS,), jnp.float32),
        pltpu.VMEM((SLOT,), jnp.int32),
        pltpu.VMEM((SLOTR, CH), jnp.int32),
        pltpu.VMEM((BLKE,), jnp.int32),
        pltpu.VMEM((BLKE,), jnp.int32),
        pltpu.VMEM((BLKE,), jnp.int32),
        pltpu.VMEM((BLKE,), jnp.int32),
        pltpu.VMEM((CH,), jnp.float32),
        pltpu.VMEM((DPT,), jnp.float32),
        pltpu.VMEM((16,), jnp.int32),
        pltpu.VMEM((PT,), jnp.float32),
        pltpu.VMEM((64, D), jnp.float32),
        pltpu.VMEM((64, D), jnp.float32),
        pltpu.VMEM((64, D), jnp.float32),
        pltpu.SemaphoreType.DMA,
        pltpu.SemaphoreType.DMA,
    ],
)
def _prep_kernel(src_hbm, dst_hbm, emb_hbm,
                 dis_hbm, disinv_hbm, u0_hbm, psrc_hbm, pdst_hbm, pcnt_hbm,
                 deg_sh, ps_loc, pd_loc, sb0, sb1, db0, db1, ones_v,
                 zeros_v, cnt_v, dis_t, ebuf, ubuf, ubuf2, sem0, sem1):
    c = lax.axis_index("c")
    s = lax.axis_index("s")
    w = c * NS + s
    lane = jnp.arange(16, dtype=jnp.int32)
    tile_e = s * KPT * CH

    @pl.loop(0, DPT // 16)
    def _(g):
        zeros_v[pl.ds(g * 16, 16)] = jnp.zeros((16,), jnp.float32)

    @pl.loop(0, CH // 16)
    def _(g):
        ones_v[pl.ds(g * 16, 16)] = jnp.ones((16,), jnp.float32)

    pltpu.sync_copy(zeros_v, deg_sh.at[pl.ds(s * DPT, DPT)])
    plsc.subcore_barrier()

    def issue(b, sb, db, sem):
        pltpu.async_copy(src_hbm.at[pl.ds(tile_e + b * BLKE, BLKE)], sb, sem)
        pltpu.async_copy(dst_hbm.at[pl.ds(tile_e + b * BLKE, BLKE)], db, sem)

    def wait(sb, db, sem):
        pltpu.make_async_copy(src_hbm.at[pl.ds(0, BLKE)], sb, sem).wait()
        pltpu.make_async_copy(dst_hbm.at[pl.ds(0, BLKE)], db, sem).wait()

    def do_block(sb, db, cnt0):
        @pl.loop(0, BLKE // 16, init_carry=cnt0)
        def cnt(g, cnt):
            v = db[pl.ds(g * 16, 16)]
            sv = sb[pl.ds(g * 16, 16)]
            loc = v - c * HALF
            m = (loc >= 0) & (loc < HALF)
            pos = plsc.cumsum(m.astype(jnp.int32))
            tgt = cnt + pos - 1
            plsc.store_scatter(
                pd_loc,
                [lax.shift_right_logical(tgt, 7), tgt & (CH - 1)],
                loc, mask=m)
            plsc.store_scatter(ps_loc, [tgt], sv, mask=m)
            return cnt + pos[15]

        return cnt

    issue(0, sb0, db0, sem0)

    @pl.loop(0, NBLK // 2, init_carry=jnp.int32(0))
    def cnt(i, cnt):
        issue(2 * i + 1, sb1, db1, sem1)
        wait(sb0, db0, sem0)
        cnt = do_block(sb0, db0, cnt)

        @pl.when(i < NBLK // 2 - 1)
        def _():
            issue(2 * i + 2, sb0, db0, sem0)

        wait(sb1, db1, sem1)
        return do_block(sb1, db1, cnt)

    for g in range(GRP * CH // 16):
        t = cnt + g * 16 + lane
        plsc.store_scatter(
            pd_loc,
            [lax.shift_right_logical(t, 7), t & (CH - 1)],
            HALF + ((g * 16 + lane) & (DR - 1)))
        plsc.store_scatter(ps_loc, [t], g * 16 + lane)

    nch = ((cnt + GRP * CH - 1) // (GRP * CH)) * GRP
    cnt_v[...] = jnp.full((16,), nch, jnp.int32)
    pltpu.sync_copy(ps_loc, psrc_hbm.at[pl.ds(w * SLOT, SLOT)])
    pltpu.sync_copy(pd_loc, pdst_hbm.at[pl.ds(w * SLOTR, SLOTR)])
    pltpu.sync_copy(cnt_v, pcnt_hbm.at[pl.ds(w * 16, 16)])

    @pl.loop(0, nch)
    def _(kk):
        pltpu.async_copy(ones_v, deg_sh.at[pd_loc.at[kk]], sem0, add=True)

    @pl.loop(0, nch)
    def _(kk):
        pltpu.make_async_copy(ones_v, deg_sh.at[pd_loc.at[kk]], sem0).wait()

    plsc.subcore_barrier()

    gbase = c * HALF + s * PT
    pltpu.sync_copy(deg_sh.at[pl.ds(s * PT, PT)], dis_t)
    pltpu.sync_copy(deg_sh.at[pl.ds(s * PT, PT)], zeros_v.at[pl.ds(0, PT)])

    @pl.loop(0, PT // 16)
    def _(k):
        r = _rsqrt16(dis_t[pl.ds(k * 16, 16)])
        dis_t[pl.ds(k * 16, 16)] = r
        zeros_v[pl.ds(k * 16, 16)] = r * zeros_v[pl.ds(k * 16, 16)]

    pltpu.sync_copy(dis_t, dis_hbm.at[pl.ds(gbase, PT)])
    pltpu.sync_copy(zeros_v.at[pl.ds(0, PT)], disinv_hbm.at[pl.ds(gbase, PT)])

    def u0_wait(stg, sem):
        pltpu.make_async_copy(stg, u0_hbm.at[pl.ds(gbase, 64)], sem).wait()

    def u0_chunk(k, stg, sem):
        pltpu.sync_copy(emb_hbm.at[pl.ds(gbase + k * 64, 64)], ebuf)
        for h in range(4):
            dv = dis_t[pl.ds(k * 64 + h * 16, 16)]
            for r in range(16):
                b = jnp.full((16,), dv[r], jnp.float32)
                for q in range(D // 16):
                    stg[h * 16 + r, pl.ds(q * 16, 16)] = (
                        ebuf[h * 16 + r, pl.ds(q * 16, 16)] * b)
        pltpu.async_copy(stg, u0_hbm.at[pl.ds(gbase + k * 64, 64)], sem)

    @pl.loop(0, PT // 128)
    def _(j):
        @pl.when(j > 0)
        def _():
            u0_wait(ubuf, sem0)

        u0_chunk(2 * j, ubuf, sem0)

        @pl.when(j > 0)
        def _():
            u0_wait(ubuf2, sem1)

        u0_chunk(2 * j + 1, ubuf2, sem1)

    u0_wait(ubuf, sem0)
    u0_chunk(PT // 64 - 1, ubuf, sem0)
    u0_wait(ubuf2, sem1)
    u0_wait(ubuf, sem0)


def _make_layer_kernel():
    def body(u_hbm, psrc_hbm, pdst_hbm, pcnt_hbm, dis_hbm, uo_hbm, *scratch):
        (acc_sh, sblk0, sblk1, dblk0, dblk1, cnt_v,
         rows0, rows1, rows2, dis64,
         isem0, isem1, gsem0, gsem1, gsem2) = scratch

        c = lax.axis_index("c")
        s = lax.axis_index("s")
        w = c * NS + s
        slot_e = w * SLOT
        slot_r = w * SLOTR

        pltpu.sync_copy(pcnt_hbm.at[pl.ds(w * 16, 16)], cnt_v)
        nch = cnt_v[...][0]
        nb2 = nch // GRP

        @pl.loop(0, CH)
        def _(r):
            @pl.loop(0, D // 16)
            def _(q):
                rows0[r, pl.ds(q * 16, 16)] = jnp.zeros((16,), jnp.float32)

        @pl.loop(0, ZPT // CH)
        def _(k):
            pltpu.async_copy(rows0, acc_sh.at[pl.ds(s * ZPT + k * CH, CH)],
                             isem0)

        pltpu.sync_copy(rows0.at[pl.ds(0, ZPT % CH)],
                        acc_sh.at[pl.ds(s * ZPT + (ZPT // CH) * CH,
                                        ZPT % CH)])

        @pl.loop(0, ZPT // CH)
        def _(k):
            pltpu.make_async_copy(rows0,
                                  acc_sh.at[pl.ds(s * ZPT, CH)], isem0).wait()

        plsc.subcore_barrier()

        def issue_blk(b, sb, db, sem):
            pltpu.async_copy(psrc_hbm.at[pl.ds(slot_e + b * LBE, LBE)],
                             sb, sem)
            pltpu.async_copy(pdst_hbm.at[pl.ds(slot_r + b * LBC, LBC)],
                             db, sem)

        def wait_blk(sb, db, sem):
            pltpu.make_async_copy(psrc_hbm.at[pl.ds(0, LBE)], sb, sem).wait()
            pltpu.make_async_copy(pdst_hbm.at[pl.ds(0, LBC)], db, sem).wait()

        def start_g(sb, t, rbuf, sem):
            pltpu.async_copy(u_hbm.at[sb.at[pl.ds(t * CH, CH)]], rbuf, sem)

        def fin(db, t, rbuf, sem):
            pltpu.make_async_copy(u_hbm.at[pl.ds(0, CH)], rbuf, sem).wait()
            pltpu.sync_copy(rbuf, acc_sh.at[db.at[t]], add=True)

        @pl.when(nch > 0)
        def _():
            issue_blk(0, sblk0, dblk0, isem0)
            issue_blk(1, sblk1, dblk1, isem1)
            wait_blk(sblk0, dblk0, isem0)
            start_g(sblk0, 0, rows0, gsem0)
            start_g(sblk0, 1, rows1, gsem1)

            @pl.loop(0, nb2)
            def _(i):
                last = i == nb2 - 1
                start_g(sblk0, 2, rows2, gsem2)
                fin(dblk0, 0, rows0, gsem0)

                wait_blk(sblk1, dblk1, isem1)
                start_g(sblk1, 0, rows0, gsem0)
                fin(dblk0, 1, rows1, gsem1)

                start_g(sblk1, 1, rows1, gsem1)
                fin(dblk0, 2, rows2, gsem2)

                @pl.when(~last)
                def _():
                    issue_blk(2 * i + 2, sblk0, dblk0, isem0)

                start_g(sblk1, 2, rows2, gsem2)
                fin(dblk1, 0, rows0, gsem0)

                @pl.when(~last)
                def _():
                    wait_blk(sblk0, dblk0, isem0)
                    start_g(sblk0, 0, rows0, gsem0)

                fin(dblk1, 1, rows1, gsem1)

                @pl.when(~last)
                def _():
                    start_g(sblk0, 1, rows1, gsem1)

                fin(dblk1, 2, rows2, gsem2)

                @pl.when(~last)
                def _():
                    issue_blk(2 * i + 3, sblk1, dblk1, isem1)

        plsc.subcore_barrier()

        gbase = c * HALF + s * PT
        lbase = s * PT

        def co_read(k):
            pltpu.sync_copy(acc_sh.at[pl.ds(lbase + k * 64, 64)],
                            rows0.at[pl.ds(0, 64)])
            pltpu.sync_copy(dis_hbm.at[pl.ds(gbase + k * 64, 64)], dis64)

        def co_wait_write(stg, sem):
            pltpu.make_async_copy(stg.at[pl.ds(0, 64)],
                                  uo_hbm.at[pl.ds(gbase, 64)], sem).wait()

        def co_chunk(k, stg, sem):
            for h in range(4):
                dv = dis64[pl.ds(h * 16, 16)]
                for r in range(16):
                    b = jnp.full((16,), dv[r], jnp.float32)
                    b2 = b * b
                    for q in range(D // 16):
                        stg[h * 16 + r, pl.ds(q * 16, 16)] = (
                            rows0[h * 16 + r, pl.ds(q * 16, 16)] * b2)
            pltpu.async_copy(stg.at[pl.ds(0, 64)],
                             uo_hbm.at[pl.ds(gbase + k * 64, 64)], sem)

        @pl.loop(0, PT // 128)
        def _(j):
            k0 = 2 * j
            co_read(k0)

            @pl.when(j > 0)
            def _():
                co_wait_write(rows1, gsem0)

            co_chunk(k0, rows1, gsem0)
            co_read(k0 + 1)

            @pl.when(j > 0)
            def _():
                co_wait_write(rows2, gsem1)

            co_chunk(k0 + 1, rows2, gsem1)

        co_read(PT // 64 - 1)
        co_wait_write(rows1, gsem0)
        co_chunk(PT // 64 - 1, rows1, gsem0)
        co_wait_write(rows2, gsem1)
        co_wait_write(rows1, gsem0)

    return pl.kernel(
        body,
        out_type=jax.ShapeDtypeStruct((NP, D), jnp.float32),
        mesh=_mesh,
        compiler_params=_params,
        scratch_types=[
            pltpu.VMEM_SHARED((ACC_ROWS, D), jnp.float32),
            pltpu.VMEM((LBE,), jnp.int32),
            pltpu.VMEM((LBE,), jnp.int32),
            pltpu.VMEM((LBC, CH), jnp.int32),
            pltpu.VMEM((LBC, CH), jnp.int32),
            pltpu.VMEM((16,), jnp.int32),
            pltpu.VMEM((CH, D), jnp.float32),
            pltpu.VMEM((CH, D), jnp.float32),
            pltpu.VMEM((CH, D), jnp.float32),
            pltpu.VMEM((64,), jnp.float32),
            pltpu.SemaphoreType.DMA,
            pltpu.SemaphoreType.DMA,
            pltpu.SemaphoreType.DMA,
            pltpu.SemaphoreType.DMA,
            pltpu.SemaphoreType.DMA,
        ],
    )


_layer_kernel = _make_layer_kernel()

_BLK = 512


def _avg_body(e_ref, u1_ref, u2_ref, u3_ref, u4_ref, di_ref, o_ref):
    usum = u1_ref[...] + u2_ref[...] + u3_ref[...] + u4_ref[...]
    o_ref[...] = (e_ref[...] + usum * di_ref[...]) * jnp.float32(
        1.0 / (L + 1) ** 2)


_avg_kernel = pl.pallas_call(
    _avg_body,
    out_shape=jax.ShapeDtypeStruct((NP, D), jnp.float32),
    grid=(NP // _BLK,),
    in_specs=[pl.BlockSpec((_BLK, D), lambda i: (i, 0))] * 5
    + [pl.BlockSpec((_BLK, 1), lambda i: (i, 0))],
    out_specs=pl.BlockSpec((_BLK, D), lambda i: (i, 0)),
)


def kernel(emb_users, emb_items, edge_index):
    emb = jnp.concatenate([emb_users, emb_items], axis=0)
    emb = jnp.pad(emb, ((0, NP - N), (0, 0)))
    src = jnp.pad(edge_index[0], (0, EP - E))
    dst = jnp.pad(edge_index[1], (0, EP - E),
                  constant_values=jnp.int32(NP))

    dis, disinv, u, psrc, pdst, pcnt = _prep_kernel(src, dst, emb)
    us = []
    for _ in range(L):
        u = _layer_kernel(u, psrc, pdst, pcnt, dis)
        us.append(u)

    out = _avg_kernel(emb, *us, disinv[:, None])
    return (out[:NU], emb_users, out[NU:N], emb_items)
